# baseline (speedup 1.0000x reference)
import jax
import jax.numpy as jnp
from jax.experimental import pallas as pl
from jax.experimental.pallas import tpu as pltpu

_B, _H, _W, _F = 16, 32, 32, 256


def _pos_body(row_ref, col_ref, out_ref, slab, sem):
    col = col_ref[0:_W, :]
    row = row_ref[0:_H, :]
    x_part = jnp.broadcast_to(col[None, None, :, :], (1, _H, _W, _F))
    y_part = jnp.broadcast_to(row[None, :, None, :], (1, _H, _W, _F))
    slab[...] = jnp.concatenate([x_part, y_part], axis=-1)
    copies = [
        pltpu.make_async_copy(
            slab.at[:, pl.ds(h, _H // 4)],
            out_ref.at[pl.ds(b, 1), pl.ds(h, _H // 4)],
            sem,
        )
        for b in range(_B)
        for h in range(0, _H, _H // 4)
    ]
    for c in copies:
        c.start()
    for c in copies:
        c.wait()


def kernel(img, row_embed, col_embed):
    del img
    out_shape = jax.ShapeDtypeStruct((_B, _H, _W, 2 * _F), jnp.float32)
    return pl.pallas_call(
        _pos_body,
        in_specs=[
            pl.BlockSpec(memory_space=pltpu.VMEM),
            pl.BlockSpec(memory_space=pltpu.VMEM),
        ],
        out_specs=pl.BlockSpec(memory_space=pl.ANY),
        out_shape=out_shape,
        scratch_shapes=[
            pltpu.VMEM((1, _H, _W, 2 * _F), jnp.float32),
            pltpu.SemaphoreType.DMA,
        ],
    )(row_embed, col_embed)


# speedup vs baseline: 1.0047x; 1.0047x over previous
import jax
import jax.numpy as jnp
from jax.experimental import pallas as pl
from jax.experimental.pallas import tpu as pltpu

_B, _H, _W, _F = 16, 32, 32, 256


def _pos_body(row_ref, col_ref, out_ref, slab, sem):
    col = col_ref[0:_W, :]
    row = row_ref[0:_H, :]
    x_part = jnp.broadcast_to(col[None, None, :, :], (1, _H, _W, _F))
    y_part = jnp.broadcast_to(row[None, :, None, :], (1, _H, _W, _F))
    slab[...] = jnp.concatenate([x_part, y_part], axis=-1)
    copies = [
        pltpu.make_async_copy(
            slab.at[:, pl.ds(h, _H // 8)],
            out_ref.at[pl.ds(b, 1), pl.ds(h, _H // 8)],
            sem,
        )
        for b in range(_B)
        for h in range(0, _H, _H // 8)
    ]
    for c in copies:
        c.start()
    for c in copies:
        c.wait()


def kernel(img, row_embed, col_embed):
    del img
    out_shape = jax.ShapeDtypeStruct((_B, _H, _W, 2 * _F), jnp.float32)
    return pl.pallas_call(
        _pos_body,
        in_specs=[
            pl.BlockSpec(memory_space=pltpu.VMEM),
            pl.BlockSpec(memory_space=pltpu.VMEM),
        ],
        out_specs=pl.BlockSpec(memory_space=pl.ANY),
        out_shape=out_shape,
        scratch_shapes=[
            pltpu.VMEM((1, _H, _W, 2 * _F), jnp.float32),
            pltpu.SemaphoreType.DMA,
        ],
    )(row_embed, col_embed)


# speedup vs baseline: 1.0065x; 1.0018x over previous
import jax
import jax.numpy as jnp
from jax.experimental import pallas as pl
from jax.experimental.pallas import tpu as pltpu

_B, _H, _W, _F = 16, 32, 32, 256


def _pos_body(row_ref, col_ref, out_ref, slab, sem):
    col = col_ref[0:_W, :]
    row = row_ref[0:_H, :]
    x_part = jnp.broadcast_to(col[None, None, :, :], (1, _H, _W, _F))
    y_part = jnp.broadcast_to(row[None, :, None, :], (1, _H, _W, _F))
    slab[...] = jnp.concatenate([x_part, y_part], axis=-1)
    copies = [
        pltpu.make_async_copy(
            slab.at[:, pl.ds(h, _H // 2)],
            out_ref.at[pl.ds(b, 1), pl.ds(h, _H // 2)],
            sem,
        )
        for b in range(_B)
        for h in (0, _H // 2)
    ]
    for c in copies:
        c.start()
    for c in copies:
        c.wait()


def kernel(img, row_embed, col_embed):
    del img
    out_shape = jax.ShapeDtypeStruct((_B, _H, _W, 2 * _F), jnp.float32)
    return pl.pallas_call(
        _pos_body,
        in_specs=[
            pl.BlockSpec(memory_space=pltpu.VMEM),
            pl.BlockSpec(memory_space=pltpu.VMEM),
        ],
        out_specs=pl.BlockSpec(memory_space=pl.ANY),
        out_shape=out_shape,
        scratch_shapes=[
            pltpu.VMEM((1, _H, _W, 2 * _F), jnp.float32),
            pltpu.SemaphoreType.DMA,
        ],
    )(row_embed, col_embed)
